# baseline (device time: 67265 ns/iter reference)
import jax
import jax.numpy as jnp
from jax import lax
from jax.experimental import pallas as pl
from jax.experimental.pallas import tpu as pltpu

N_DEV = 4
SQ = 1024
SKV = 1024
H_PER = 8
DH = 128
HD = H_PER * DH
DM = 1024
CHUNK = SQ // N_DEV
SCALE = 0.08838834764831843
N_HOPS = 2 * (N_DEV - 1)


HALF = DM // 2
BW = 512
GW = 128


def _body(x_ref, wq_hbm, k_in, v_in, wo_hbm, out_ref,
          acc_ref, x16_ref, k16_ref, v16_ref,
          wq32_ref, wq16_ref, wo32_ref, wo16_ref,
          comm_f, comm_b,
          send_f, recv_f, send_b, recv_b, cp_sems):
    my = lax.axis_index("i")
    right = lax.rem(my + 1, N_DEV)
    left = lax.rem(my + N_DEV - 1, N_DEV)

    barrier_sem = pltpu.get_barrier_semaphore()
    for nbr in (left, right):
        pl.semaphore_signal(barrier_sem, inc=1, device_id=(nbr,),
                            device_id_type=pl.DeviceIdType.MESH)

    w_off = pl.multiple_of(my * HD, 256)
    wq_dma = pltpu.make_async_copy(
        wq_hbm.at[:, pl.ds(w_off, HD)], wq32_ref, cp_sems.at[0])
    wo_dma = pltpu.make_async_copy(
        wo_hbm.at[pl.ds(w_off, HD), :], wo32_ref, cp_sems.at[1])
    wq_dma.start()
    wo_dma.start()
    x16_ref[...] = x_ref[...].astype(jnp.bfloat16)
    k16_ref[...] = k_in[...].astype(jnp.bfloat16)
    v16_ref[...] = v_in[...].astype(jnp.bfloat16)
    wq_dma.wait()
    wq16_ref[...] = wq32_ref[...].astype(jnp.bfloat16)
    wo_dma.wait()
    wo16_ref[...] = wo32_ref[...].astype(jnp.bfloat16)

    def compute_chunk(c):
        r0 = c * CHUNK
        bs = jnp.minimum(jnp.maximum(r0 - 128, 0), SKV - BW)
        bs = pl.multiple_of(bs, 128)
        qc = (jnp.dot(x16_ref[pl.ds(r0, CHUNK), :], wq16_ref[...],
                      preferred_element_type=jnp.float32)
              * SCALE).astype(jnp.bfloat16)
        qi = lax.broadcasted_iota(jnp.int32, (CHUNK, BW), 0) + r0
        kiB = lax.broadcasted_iota(jnp.int32, (CHUNK, BW), 1) + bs
        mB = ((jnp.abs(qi - kiB) <= 128) | (kiB < 32) | (qi < 32)
              ).astype(jnp.float32)
        kiG = lax.broadcasted_iota(jnp.int32, (CHUNK, GW), 1)
        mG = ((kiG < 32) & (bs >= 128)).astype(jnp.float32)
        acc = None
        for h in range(H_PER):
            hs = slice(h * DH, (h + 1) * DH)
            qh = qc[:, hs]
            sB = lax.dot_general(qh, k16_ref[pl.ds(bs, BW), hs],
                                 (((1,), (1,)), ((), ())),
                                 preferred_element_type=jnp.float32)
            eB = jnp.exp(sB) * mB
            sG = lax.dot_general(qh, k16_ref[0:GW, hs],
                                 (((1,), (1,)), ((), ())),
                                 preferred_element_type=jnp.float32)
            eG = jnp.exp(sG) * mG
            denom = (jnp.sum(eB, axis=1, keepdims=True)
                     + jnp.sum(eG, axis=1, keepdims=True))
            ctx_h = (jnp.dot(eB.astype(jnp.bfloat16), v16_ref[pl.ds(bs, BW), hs],
                             preferred_element_type=jnp.float32)
                     + jnp.dot(eG.astype(jnp.bfloat16), v16_ref[0:GW, hs],
                               preferred_element_type=jnp.float32)) / denom
            part = jnp.dot(ctx_h.astype(jnp.bfloat16),
                           wo16_ref[hs, :],
                           preferred_element_type=jnp.float32)
            acc = part if acc is None else acc + part
        acc_ref[pl.ds(r0, CHUNK), :] = acc

        @pl.when(c == 0)
        def _global_rows():
            q0 = (jnp.dot(x16_ref[0:32, :], wq16_ref[...],
                          preferred_element_type=jnp.float32)
                  * SCALE).astype(jnp.bfloat16)
            acc0 = None
            for h in range(H_PER):
                hs = slice(h * DH, (h + 1) * DH)
                s = lax.dot_general(q0[:, hs], k16_ref[:, hs],
                                    (((1,), (1,)), ((), ())),
                                    preferred_element_type=jnp.float32)
                e = jnp.exp(s)
                ctx_h = jnp.dot(e.astype(jnp.bfloat16), v16_ref[:, hs],
                                preferred_element_type=jnp.float32
                                ) / jnp.sum(e, axis=1, keepdims=True)
                p = jnp.dot(ctx_h.astype(jnp.bfloat16),
                            wo16_ref[hs, :],
                            preferred_element_type=jnp.float32)
                acc0 = p if acc0 is None else acc0 + p
            acc_ref[0:32, :] = acc0

    def hop_rdma(hop, fwd):
        s_slot = hop % N_DEV
        r_slot = (hop + 1) % N_DEV
        comm = comm_f if fwd else comm_b
        return pltpu.make_async_remote_copy(
            src_ref=comm.at[s_slot], dst_ref=comm.at[r_slot],
            send_sem=(send_f if fwd else send_b).at[hop],
            recv_sem=(recv_f if fwd else recv_b).at[hop],
            device_id=(right if fwd else left,),
            device_id_type=pl.DeviceIdType.MESH,
        )

    def rs_accum(hop, fwd):
        r_slot = (hop + 1) % N_DEV
        if fwd:
            c = lax.rem(my + (2 * N_DEV - hop - 1), N_DEV)
            comm_f[r_slot, :, :] = (
                comm_f[r_slot, :, :].astype(jnp.float32)
                + acc_ref[pl.ds(c * CHUNK, CHUNK), 0:HALF]
            ).astype(jnp.bfloat16)
        else:
            c = lax.rem(my + hop + 1, N_DEV)
            comm_b[r_slot, :, :] = (
                comm_b[r_slot, :, :].astype(jnp.float32)
                + acc_ref[pl.ds(c * CHUNK, CHUNK), HALF:DM]
            ).astype(jnp.bfloat16)
        return c, r_slot

    def out_store(c, slot, fwd):
        if fwd:
            out_ref[pl.ds(c * CHUNK, CHUNK), 0:HALF] = (
                comm_f[slot, :, :].astype(jnp.float32))
        else:
            out_ref[pl.ds(c * CHUNK, CHUNK), HALF:DM] = (
                comm_b[slot, :, :].astype(jnp.float32))

    c_my = my
    c_m1 = lax.rem(my + N_DEV - 1, N_DEV)
    c_p1 = lax.rem(my + 1, N_DEV)
    c_m2 = lax.rem(my + N_DEV - 2, N_DEV)

    import os
    if os.environ.get("NO_COMM"):
        compute_chunk(c_my)
        compute_chunk(c_m1)
        compute_chunk(c_p1)
        compute_chunk(c_m2)
        pl.semaphore_wait(barrier_sem, 2)
        out_ref[...] = acc_ref[...]
        return

    compute_chunk(c_my)
    pl.semaphore_wait(barrier_sem, 2)
    comm_f[0, :, :] = acc_ref[pl.ds(my * CHUNK, CHUNK), 0:HALF].astype(jnp.bfloat16)
    comm_b[0, :, :] = acc_ref[pl.ds(my * CHUNK, CHUNK), HALF:DM].astype(jnp.bfloat16)
    f0 = hop_rdma(0, True)
    b0 = hop_rdma(0, False)
    f0.start()
    b0.start()

    compute_chunk(c_m1)
    f0.wait()
    rs_accum(0, True)
    f1 = hop_rdma(1, True)
    f1.start()

    compute_chunk(c_p1)
    b0.wait()
    rs_accum(0, False)
    b1 = hop_rdma(1, False)
    b1.start()

    compute_chunk(c_m2)
    f1.wait()
    rs_accum(1, True)
    f2 = hop_rdma(2, True)
    f2.start()
    b1.wait()
    rs_accum(1, False)
    b2 = hop_rdma(2, False)
    b2.start()

    f2.wait()
    cf, sf = rs_accum(2, True)
    out_store(cf, sf, True)
    fa0 = hop_rdma(3, True)
    fa0.start()
    b2.wait()
    cb, sb = rs_accum(2, False)
    out_store(cb, sb, False)
    ba0 = hop_rdma(3, False)
    ba0.start()

    fa0.wait()
    out_store(my, 0, True)
    fa1 = hop_rdma(4, True)
    fa1.start()
    ba0.wait()
    out_store(my, 0, False)
    ba1 = hop_rdma(4, False)
    ba1.start()

    fa1.wait()
    out_store(c_m1, 1, True)
    fa2 = hop_rdma(5, True)
    fa2.start()
    ba1.wait()
    out_store(c_p1, 1, False)
    ba2 = hop_rdma(5, False)
    ba2.start()

    fa2.wait()
    out_store(c_m2, 2, True)
    ba2.wait()
    out_store(c_m2, 2, False)


def kernel(x, Wq, K_ext, V_ext, Wo):
    x2 = x[0]
    kflat = K_ext.reshape(SKV, HD)
    vflat = V_ext.reshape(SKV, HD)

    out = pl.pallas_call(
        _body,
        out_shape=jax.ShapeDtypeStruct((SQ, DM), jnp.float32),
        in_specs=[
            pl.BlockSpec(memory_space=pltpu.VMEM),
            pl.BlockSpec(memory_space=pl.ANY),
            pl.BlockSpec(memory_space=pltpu.VMEM),
            pl.BlockSpec(memory_space=pltpu.VMEM),
            pl.BlockSpec(memory_space=pl.ANY),
        ],
        out_specs=pl.BlockSpec(memory_space=pltpu.VMEM),
        scratch_shapes=[
            pltpu.VMEM((SQ, DM), jnp.float32),
            pltpu.VMEM((SQ, DM), jnp.bfloat16),
            pltpu.VMEM((SKV, HD), jnp.bfloat16),
            pltpu.VMEM((SKV, HD), jnp.bfloat16),
            pltpu.VMEM((DM, HD), jnp.float32),
            pltpu.VMEM((DM, HD), jnp.bfloat16),
            pltpu.VMEM((HD, DM), jnp.float32),
            pltpu.VMEM((HD, DM), jnp.bfloat16),
            pltpu.VMEM((N_DEV, CHUNK, HALF), jnp.bfloat16),
            pltpu.VMEM((N_DEV, CHUNK, HALF), jnp.bfloat16),
            pltpu.SemaphoreType.DMA((N_HOPS,)),
            pltpu.SemaphoreType.DMA((N_HOPS,)),
            pltpu.SemaphoreType.DMA((N_HOPS,)),
            pltpu.SemaphoreType.DMA((N_HOPS,)),
            pltpu.SemaphoreType.DMA((2,)),
        ],
        compiler_params=pltpu.CompilerParams(
            collective_id=0, vmem_limit_bytes=64 * 1024 * 1024),
    )(x2, Wq, kflat, vflat, Wo)
    return out[None]


# device time: 56013 ns/iter; 1.2009x vs baseline; 1.2009x over previous
import jax
import jax.numpy as jnp
from jax import lax
from jax.experimental import pallas as pl
from jax.experimental.pallas import tpu as pltpu

N_DEV = 4
SQ = 1024
SKV = 1024
H_PER = 8
DH = 128
HD = H_PER * DH
DM = 1024
CHUNK = SQ // N_DEV
SCALE = 0.08838834764831843
N_HOPS = 2 * (N_DEV - 1)


HALF = DM // 2
BW = 512
GW = 128


def _body(x_ref, wq_hbm, k_in, v_in, wo_hbm, out_ref,
          acc_ref, x16_ref, k16_ref, v16_ref,
          wq32_ref, wq16_ref, wo32_ref, wo16_ref,
          comm_f, comm_b,
          send_f, recv_f, send_b, recv_b, cp_sems):
    my = lax.axis_index("i")
    right = lax.rem(my + 1, N_DEV)
    left = lax.rem(my + N_DEV - 1, N_DEV)

    barrier_sem = pltpu.get_barrier_semaphore()
    for nbr in (left, right):
        pl.semaphore_signal(barrier_sem, inc=1, device_id=(nbr,),
                            device_id_type=pl.DeviceIdType.MESH)

    w_off = pl.multiple_of(my * HD, 256)
    wq_dma = pltpu.make_async_copy(
        wq_hbm.at[:, pl.ds(w_off, HD)], wq32_ref, cp_sems.at[0])
    wo_dma = pltpu.make_async_copy(
        wo_hbm.at[pl.ds(w_off, HD), :], wo32_ref, cp_sems.at[1])
    wq_dma.start()
    wo_dma.start()
    x16_ref[...] = x_ref[...].astype(jnp.bfloat16)
    k16_ref[...] = k_in[...].astype(jnp.bfloat16)
    v16_ref[...] = v_in[...].astype(jnp.bfloat16)
    wq_dma.wait()
    wq16_ref[...] = wq32_ref[...].astype(jnp.bfloat16)
    wo_dma.wait()
    wo16_ref[...] = wo32_ref[...].astype(jnp.bfloat16)

    def compute_chunk(c):
        r0 = c * CHUNK
        bs = jnp.minimum(jnp.maximum(r0 - 128, 0), SKV - BW)
        bs = pl.multiple_of(bs, 128)
        qc = (jnp.dot(x16_ref[pl.ds(r0, CHUNK), :], wq16_ref[...],
                      preferred_element_type=jnp.float32)
              * SCALE).astype(jnp.bfloat16)
        qi = lax.broadcasted_iota(jnp.int32, (CHUNK, BW), 0) + r0
        kiB = lax.broadcasted_iota(jnp.int32, (CHUNK, BW), 1) + bs
        mB = ((jnp.abs(qi - kiB) <= 128) | (kiB < 32) | (qi < 32)
              ).astype(jnp.float32)
        kiG = lax.broadcasted_iota(jnp.int32, (CHUNK, GW), 1)
        mG = ((kiG < 32) & (bs >= 128)).astype(jnp.float32)
        acc = None
        for h in range(H_PER):
            hs = slice(h * DH, (h + 1) * DH)
            qh = qc[:, hs]
            sB = lax.dot_general(qh, k16_ref[pl.ds(bs, BW), hs],
                                 (((1,), (1,)), ((), ())),
                                 preferred_element_type=jnp.float32)
            eB = jnp.exp(sB) * mB
            sG = lax.dot_general(qh, k16_ref[0:GW, hs],
                                 (((1,), (1,)), ((), ())),
                                 preferred_element_type=jnp.float32)
            eG = jnp.exp(sG) * mG
            denom = (jnp.sum(eB, axis=1, keepdims=True)
                     + jnp.sum(eG, axis=1, keepdims=True))
            ctx_h = (jnp.dot(eB.astype(jnp.bfloat16), v16_ref[pl.ds(bs, BW), hs],
                             preferred_element_type=jnp.float32)
                     + jnp.dot(eG.astype(jnp.bfloat16), v16_ref[0:GW, hs],
                               preferred_element_type=jnp.float32)) / denom
            part = jnp.dot(ctx_h.astype(jnp.bfloat16),
                           wo16_ref[hs, :],
                           preferred_element_type=jnp.float32)
            acc = part if acc is None else acc + part
        acc_ref[pl.ds(r0, CHUNK), :] = acc

        @pl.when(c == 0)
        def _global_rows():
            q0 = (jnp.dot(x16_ref[0:32, :], wq16_ref[...],
                          preferred_element_type=jnp.float32)
                  * SCALE).astype(jnp.bfloat16)
            acc0 = None
            for h in range(H_PER):
                hs = slice(h * DH, (h + 1) * DH)
                s = lax.dot_general(q0[:, hs], k16_ref[:, hs],
                                    (((1,), (1,)), ((), ())),
                                    preferred_element_type=jnp.float32)
                e = jnp.exp(s)
                ctx_h = jnp.dot(e.astype(jnp.bfloat16), v16_ref[:, hs],
                                preferred_element_type=jnp.float32
                                ) / jnp.sum(e, axis=1, keepdims=True)
                p = jnp.dot(ctx_h.astype(jnp.bfloat16),
                            wo16_ref[hs, :],
                            preferred_element_type=jnp.float32)
                acc0 = p if acc0 is None else acc0 + p
            acc_ref[0:32, :] = acc0

    def hop_rdma(hop, fwd):
        s_slot = hop % N_DEV
        r_slot = (hop + 1) % N_DEV
        comm = comm_f if fwd else comm_b
        return pltpu.make_async_remote_copy(
            src_ref=comm.at[s_slot], dst_ref=comm.at[r_slot],
            send_sem=(send_f if fwd else send_b).at[hop],
            recv_sem=(recv_f if fwd else recv_b).at[hop],
            device_id=(right if fwd else left,),
            device_id_type=pl.DeviceIdType.MESH,
        )

    def rs_accum(hop, fwd):
        r_slot = (hop + 1) % N_DEV
        if fwd:
            c = lax.rem(my + (2 * N_DEV - hop - 1), N_DEV)
            comm_f[r_slot, :, :] = (
                comm_f[r_slot, :, :].astype(jnp.float32)
                + acc_ref[pl.ds(c * CHUNK, CHUNK), 0:HALF]
            ).astype(jnp.bfloat16)
        else:
            c = lax.rem(my + hop + 1, N_DEV)
            comm_b[r_slot, :, :] = (
                comm_b[r_slot, :, :].astype(jnp.float32)
                + acc_ref[pl.ds(c * CHUNK, CHUNK), HALF:DM]
            ).astype(jnp.bfloat16)
        return c, r_slot

    def out_store(c, slot, fwd):
        if fwd:
            out_ref[pl.ds(c * CHUNK, CHUNK), 0:HALF] = (
                comm_f[slot, :, :].astype(jnp.float32))
        else:
            out_ref[pl.ds(c * CHUNK, CHUNK), HALF:DM] = (
                comm_b[slot, :, :].astype(jnp.float32))

    c_my = my
    c_m1 = lax.rem(my + N_DEV - 1, N_DEV)
    c_p1 = lax.rem(my + 1, N_DEV)
    c_m2 = lax.rem(my + N_DEV - 2, N_DEV)

    import os
    if os.environ.get("NO_COMM"):
        compute_chunk(c_my)
        compute_chunk(c_m1)
        compute_chunk(c_p1)
        compute_chunk(c_m2)
        pl.semaphore_wait(barrier_sem, 2)
        out_ref[...] = acc_ref[...]
        return

    compute_chunk(c_my)
    pl.semaphore_wait(barrier_sem, 2)
    comm_f[0, :, :] = acc_ref[pl.ds(my * CHUNK, CHUNK), 0:HALF].astype(jnp.bfloat16)
    comm_b[0, :, :] = acc_ref[pl.ds(my * CHUNK, CHUNK), HALF:DM].astype(jnp.bfloat16)
    f0 = hop_rdma(0, True)
    b0 = hop_rdma(0, False)
    f0.start()
    b0.start()

    compute_chunk(c_m1)
    f0.wait()
    rs_accum(0, True)
    f1 = hop_rdma(1, True)
    f1.start()

    compute_chunk(c_p1)
    b0.wait()
    rs_accum(0, False)
    b1 = hop_rdma(1, False)
    b1.start()

    compute_chunk(c_m2)
    f1.wait()
    rs_accum(1, True)
    f2 = hop_rdma(2, True)
    f2.start()
    b1.wait()
    rs_accum(1, False)
    b2 = hop_rdma(2, False)
    b2.start()

    f2.wait()
    cf, sf = rs_accum(2, True)
    out_store(cf, sf, True)
    fa0 = hop_rdma(3, True)
    fa0.start()
    b2.wait()
    cb, sb = rs_accum(2, False)
    out_store(cb, sb, False)
    ba0 = hop_rdma(3, False)
    ba0.start()

    fa0.wait()
    out_store(my, 0, True)
    fa1 = hop_rdma(4, True)
    fa1.start()
    ba0.wait()
    out_store(my, 0, False)
    ba1 = hop_rdma(4, False)
    ba1.start()

    fa1.wait()
    out_store(c_m1, 1, True)
    fa2 = hop_rdma(5, True)
    fa2.start()
    ba1.wait()
    out_store(c_p1, 1, False)
    ba2 = hop_rdma(5, False)
    ba2.start()

    fa2.wait()
    out_store(c_m2, 2, True)
    ba2.wait()
    out_store(c_m2, 2, False)


def kernel(x, Wq, K_ext, V_ext, Wo):
    x2 = x[0]
    kflat = K_ext.reshape(SKV, HD)
    vflat = V_ext.reshape(SKV, HD)

    out = pl.pallas_call(
        _body,
        out_shape=jax.ShapeDtypeStruct((SQ, DM), jnp.float32),
        in_specs=[
            pl.BlockSpec(memory_space=pltpu.VMEM),
            pl.BlockSpec(memory_space=pl.ANY),
            pl.BlockSpec(memory_space=pltpu.VMEM),
            pl.BlockSpec(memory_space=pltpu.VMEM),
            pl.BlockSpec(memory_space=pl.ANY),
        ],
        out_specs=pl.BlockSpec(memory_space=pltpu.VMEM),
        scratch_shapes=[
            pltpu.VMEM((SQ, DM), jnp.float32),
            pltpu.VMEM((SQ, DM), jnp.bfloat16),
            pltpu.VMEM((SKV, HD), jnp.bfloat16),
            pltpu.VMEM((SKV, HD), jnp.bfloat16),
            pltpu.VMEM((DM, HD), jnp.float32),
            pltpu.VMEM((DM, HD), jnp.bfloat16),
            pltpu.VMEM((HD, DM), jnp.float32),
            pltpu.VMEM((HD, DM), jnp.bfloat16),
            pltpu.VMEM((N_DEV, CHUNK, HALF), jnp.bfloat16),
            pltpu.VMEM((N_DEV, CHUNK, HALF), jnp.bfloat16),
            pltpu.SemaphoreType.DMA((N_HOPS,)),
            pltpu.SemaphoreType.DMA((N_HOPS,)),
            pltpu.SemaphoreType.DMA((N_HOPS,)),
            pltpu.SemaphoreType.DMA((N_HOPS,)),
            pltpu.SemaphoreType.DMA((2,)),
        ],
        compiler_params=pltpu.CompilerParams(collective_id=0),
    )(x2, Wq, kflat, vflat, Wo)
    return out[None]


# device time: 55131 ns/iter; 1.2201x vs baseline; 1.0160x over previous
import jax
import jax.numpy as jnp
from jax import lax
from jax.experimental import pallas as pl
from jax.experimental.pallas import tpu as pltpu

N_DEV = 4
SQ = 1024
SKV = 1024
H_PER = 8
DH = 128
HD = H_PER * DH
DM = 1024
CHUNK = SQ // N_DEV
SCALE = 0.08838834764831843
N_HOPS = 2 * (N_DEV - 1)


HALF = DM // 2
BW = 512
GW = 128


def _body(x_ref, wq_hbm, k_ref, v_ref, wo_hbm, out_ref,
          acc_ref, x16_ref,
          wq32_ref, wq16_ref, wo32_ref, wo16_ref,
          comm_f, comm_b,
          send_f, recv_f, send_b, recv_b, cp_sems):
    my = lax.axis_index("i")
    right = lax.rem(my + 1, N_DEV)
    left = lax.rem(my + N_DEV - 1, N_DEV)

    barrier_sem = pltpu.get_barrier_semaphore()
    for nbr in (left, right):
        pl.semaphore_signal(barrier_sem, inc=1, device_id=(nbr,),
                            device_id_type=pl.DeviceIdType.MESH)

    w_off = pl.multiple_of(my * HD, 256)
    wq_dma = pltpu.make_async_copy(
        wq_hbm.at[:, pl.ds(w_off, HD)], wq32_ref, cp_sems.at[0])
    wo_dma = pltpu.make_async_copy(
        wo_hbm.at[pl.ds(w_off, HD), :], wo32_ref, cp_sems.at[1])
    wq_dma.start()
    wo_dma.start()
    x16_ref[...] = x_ref[...].astype(jnp.bfloat16)
    wq_dma.wait()
    wq16_ref[...] = wq32_ref[...].astype(jnp.bfloat16)
    wo_dma.wait()
    wo16_ref[...] = wo32_ref[...].astype(jnp.bfloat16)

    def compute_chunk(c):
        r0 = c * CHUNK
        bs = jnp.minimum(jnp.maximum(r0 - 128, 0), SKV - BW)
        bs = pl.multiple_of(bs, 128)
        qc = (jnp.dot(x16_ref[pl.ds(r0, CHUNK), :], wq16_ref[...],
                      preferred_element_type=jnp.float32)
              * SCALE).astype(jnp.bfloat16)
        qi = lax.broadcasted_iota(jnp.int32, (CHUNK, BW), 0) + r0
        kiB = lax.broadcasted_iota(jnp.int32, (CHUNK, BW), 1) + bs
        mB = ((jnp.abs(qi - kiB) <= 128) | (kiB < 32) | (qi < 32)
              ).astype(jnp.float32)
        kiG = lax.broadcasted_iota(jnp.int32, (CHUNK, GW), 1)
        mG = ((kiG < 32) & (bs >= 128)).astype(jnp.float32)
        acc = None
        for h in range(H_PER):
            hs = slice(h * DH, (h + 1) * DH)
            qh = qc[:, hs]
            sB = lax.dot_general(qh, k_ref[h, pl.ds(bs, BW), :],
                                 (((1,), (1,)), ((), ())),
                                 preferred_element_type=jnp.float32)
            eB = jnp.exp(sB) * mB
            sG = lax.dot_general(qh, k_ref[h, 0:GW, :],
                                 (((1,), (1,)), ((), ())),
                                 preferred_element_type=jnp.float32)
            eG = jnp.exp(sG) * mG
            denom = (jnp.sum(eB, axis=1, keepdims=True)
                     + jnp.sum(eG, axis=1, keepdims=True))
            ctx_h = (jnp.dot(eB.astype(jnp.bfloat16), v_ref[h, pl.ds(bs, BW), :],
                             preferred_element_type=jnp.float32)
                     + jnp.dot(eG.astype(jnp.bfloat16), v_ref[h, 0:GW, :],
                               preferred_element_type=jnp.float32)) / denom
            part = jnp.dot(ctx_h.astype(jnp.bfloat16),
                           wo16_ref[hs, :],
                           preferred_element_type=jnp.float32)
            acc = part if acc is None else acc + part
        acc_ref[pl.ds(r0, CHUNK), :] = acc

        @pl.when(c == 0)
        def _global_rows():
            q0 = (jnp.dot(x16_ref[0:32, :], wq16_ref[...],
                          preferred_element_type=jnp.float32)
                  * SCALE).astype(jnp.bfloat16)
            acc0 = None
            for h in range(H_PER):
                hs = slice(h * DH, (h + 1) * DH)
                s = lax.dot_general(q0[:, hs], k_ref[h],
                                    (((1,), (1,)), ((), ())),
                                    preferred_element_type=jnp.float32)
                e = jnp.exp(s)
                ctx_h = jnp.dot(e.astype(jnp.bfloat16), v_ref[h],
                                preferred_element_type=jnp.float32
                                ) / jnp.sum(e, axis=1, keepdims=True)
                p = jnp.dot(ctx_h.astype(jnp.bfloat16),
                            wo16_ref[hs, :],
                            preferred_element_type=jnp.float32)
                acc0 = p if acc0 is None else acc0 + p
            acc_ref[0:32, :] = acc0

    def hop_rdma(hop, fwd):
        s_slot = hop % N_DEV
        r_slot = (hop + 1) % N_DEV
        comm = comm_f if fwd else comm_b
        return pltpu.make_async_remote_copy(
            src_ref=comm.at[s_slot], dst_ref=comm.at[r_slot],
            send_sem=(send_f if fwd else send_b).at[hop],
            recv_sem=(recv_f if fwd else recv_b).at[hop],
            device_id=(right if fwd else left,),
            device_id_type=pl.DeviceIdType.MESH,
        )

    def rs_accum(hop, fwd):
        r_slot = (hop + 1) % N_DEV
        if fwd:
            c = lax.rem(my + (2 * N_DEV - hop - 1), N_DEV)
            comm_f[r_slot, :, :] = (
                comm_f[r_slot, :, :].astype(jnp.float32)
                + acc_ref[pl.ds(c * CHUNK, CHUNK), 0:HALF]
            ).astype(jnp.bfloat16)
        else:
            c = lax.rem(my + hop + 1, N_DEV)
            comm_b[r_slot, :, :] = (
                comm_b[r_slot, :, :].astype(jnp.float32)
                + acc_ref[pl.ds(c * CHUNK, CHUNK), HALF:DM]
            ).astype(jnp.bfloat16)
        return c, r_slot

    def out_store(c, slot, fwd):
        if fwd:
            out_ref[pl.ds(c * CHUNK, CHUNK), 0:HALF] = (
                comm_f[slot, :, :].astype(jnp.float32))
        else:
            out_ref[pl.ds(c * CHUNK, CHUNK), HALF:DM] = (
                comm_b[slot, :, :].astype(jnp.float32))

    c_my = my
    c_m1 = lax.rem(my + N_DEV - 1, N_DEV)
    c_p1 = lax.rem(my + 1, N_DEV)
    c_m2 = lax.rem(my + N_DEV - 2, N_DEV)

    import os
    if os.environ.get("NO_COMM"):
        compute_chunk(c_my)
        compute_chunk(c_m1)
        compute_chunk(c_p1)
        compute_chunk(c_m2)
        pl.semaphore_wait(barrier_sem, 2)
        out_ref[...] = acc_ref[...]
        return

    compute_chunk(c_my)
    pl.semaphore_wait(barrier_sem, 2)
    comm_f[0, :, :] = acc_ref[pl.ds(my * CHUNK, CHUNK), 0:HALF].astype(jnp.bfloat16)
    comm_b[0, :, :] = acc_ref[pl.ds(my * CHUNK, CHUNK), HALF:DM].astype(jnp.bfloat16)
    f0 = hop_rdma(0, True)
    b0 = hop_rdma(0, False)
    f0.start()
    b0.start()

    compute_chunk(c_m1)
    f0.wait()
    rs_accum(0, True)
    f1 = hop_rdma(1, True)
    f1.start()

    compute_chunk(c_p1)
    b0.wait()
    rs_accum(0, False)
    b1 = hop_rdma(1, False)
    b1.start()

    compute_chunk(c_m2)
    f1.wait()
    rs_accum(1, True)
    f2 = hop_rdma(2, True)
    f2.start()
    b1.wait()
    rs_accum(1, False)
    b2 = hop_rdma(2, False)
    b2.start()

    f2.wait()
    cf, sf = rs_accum(2, True)
    out_store(cf, sf, True)
    fa0 = hop_rdma(3, True)
    fa0.start()
    b2.wait()
    cb, sb = rs_accum(2, False)
    out_store(cb, sb, False)
    ba0 = hop_rdma(3, False)
    ba0.start()

    fa0.wait()
    out_store(my, 0, True)
    fa1 = hop_rdma(4, True)
    fa1.start()
    ba0.wait()
    out_store(my, 0, False)
    ba1 = hop_rdma(4, False)
    ba1.start()

    fa1.wait()
    out_store(c_m1, 1, True)
    fa2 = hop_rdma(5, True)
    fa2.start()
    ba1.wait()
    out_store(c_p1, 1, False)
    ba2 = hop_rdma(5, False)
    ba2.start()

    fa2.wait()
    out_store(c_m2, 2, True)
    ba2.wait()
    out_store(c_m2, 2, False)


def kernel(x, Wq, K_ext, V_ext, Wo):
    x2 = x[0]
    k = jnp.transpose(K_ext[0], (1, 0, 2)).astype(jnp.bfloat16)
    v = jnp.transpose(V_ext[0], (1, 0, 2)).astype(jnp.bfloat16)

    out = pl.pallas_call(
        _body,
        out_shape=jax.ShapeDtypeStruct((SQ, DM), jnp.float32),
        in_specs=[
            pl.BlockSpec(memory_space=pltpu.VMEM),
            pl.BlockSpec(memory_space=pl.ANY),
            pl.BlockSpec(memory_space=pltpu.VMEM),
            pl.BlockSpec(memory_space=pltpu.VMEM),
            pl.BlockSpec(memory_space=pl.ANY),
        ],
        out_specs=pl.BlockSpec(memory_space=pltpu.VMEM),
        scratch_shapes=[
            pltpu.VMEM((SQ, DM), jnp.float32),
            pltpu.VMEM((SQ, DM), jnp.bfloat16),
            pltpu.VMEM((DM, HD), jnp.float32),
            pltpu.VMEM((DM, HD), jnp.bfloat16),
            pltpu.VMEM((HD, DM), jnp.float32),
            pltpu.VMEM((HD, DM), jnp.bfloat16),
            pltpu.VMEM((N_DEV, CHUNK, HALF), jnp.bfloat16),
            pltpu.VMEM((N_DEV, CHUNK, HALF), jnp.bfloat16),
            pltpu.SemaphoreType.DMA((N_HOPS,)),
            pltpu.SemaphoreType.DMA((N_HOPS,)),
            pltpu.SemaphoreType.DMA((N_HOPS,)),
            pltpu.SemaphoreType.DMA((N_HOPS,)),
            pltpu.SemaphoreType.DMA((2,)),
        ],
        compiler_params=pltpu.CompilerParams(collective_id=0),
    )(x2, Wq, k, v, Wo)
    return out[None]


# device time: 54693 ns/iter; 1.2299x vs baseline; 1.0080x over previous
import jax
import jax.numpy as jnp
from jax import lax
from jax.experimental import pallas as pl
from jax.experimental.pallas import tpu as pltpu

N_DEV = 4
SQ = 1024
SKV = 1024
H_PER = 8
DH = 128
HD = H_PER * DH
DM = 1024
CHUNK = SQ // N_DEV
SCALE = 0.08838834764831843
N_HOPS = 2 * (N_DEV - 1)


HALF = DM // 2
BW = 512
GW = 128


def _body(x_ref, wq_hbm, k_ref, v_ref, wo_hbm, out_ref,
          acc_ref, x16_ref,
          wq32_ref, wq16_ref, wo32_ref, wo16_ref,
          comm_f, comm_b,
          send_f, recv_f, send_b, recv_b, cp_sems):
    my = lax.axis_index("i")
    right = lax.rem(my + 1, N_DEV)
    left = lax.rem(my + N_DEV - 1, N_DEV)

    barrier_sem = pltpu.get_barrier_semaphore()
    for nbr in (left, right):
        pl.semaphore_signal(barrier_sem, inc=1, device_id=(nbr,),
                            device_id_type=pl.DeviceIdType.MESH)

    w_off = pl.multiple_of(my * HD, 256)
    wq_dma = pltpu.make_async_copy(
        wq_hbm.at[:, pl.ds(w_off, HD)], wq32_ref, cp_sems.at[0])
    wo_dma = pltpu.make_async_copy(
        wo_hbm.at[pl.ds(w_off, HD), :], wo32_ref, cp_sems.at[1])
    wq_dma.start()
    wo_dma.start()
    x16_ref[...] = x_ref[...].astype(jnp.bfloat16)
    wq_dma.wait()
    wq16_ref[...] = wq32_ref[...].astype(jnp.bfloat16)
    wo_dma.wait()
    wo16_ref[...] = wo32_ref[...].astype(jnp.bfloat16)

    def compute_chunk(c):
        r0 = c * CHUNK
        bs = jnp.minimum(jnp.maximum(r0 - 128, 0), SKV - BW)
        bs = pl.multiple_of(bs, 128)
        qc = (jnp.dot(x16_ref[pl.ds(r0, CHUNK), :], wq16_ref[...],
                      preferred_element_type=jnp.float32)
              * SCALE).astype(jnp.bfloat16)
        qi = lax.broadcasted_iota(jnp.int32, (CHUNK, BW), 0) + r0
        kiB = lax.broadcasted_iota(jnp.int32, (CHUNK, BW), 1) + bs
        mB = ((jnp.abs(qi - kiB) <= 128) | (kiB < 32) | (qi < 32)
              ).astype(jnp.float32)
        kiG = lax.broadcasted_iota(jnp.int32, (CHUNK, GW), 1)
        mG = ((kiG < 32) & (bs >= 128)).astype(jnp.float32)
        acc = None
        for h in range(H_PER):
            hs = slice(h * DH, (h + 1) * DH)
            qh = qc[:, hs]
            sB = lax.dot_general(qh, k_ref[h, pl.ds(bs, BW), :],
                                 (((1,), (1,)), ((), ())),
                                 preferred_element_type=jnp.float32)
            eB = jnp.exp(sB) * mB
            sG = lax.dot_general(qh, k_ref[h, 0:GW, :],
                                 (((1,), (1,)), ((), ())),
                                 preferred_element_type=jnp.float32)
            eG = jnp.exp(sG) * mG
            denom = (jnp.sum(eB, axis=1, keepdims=True)
                     + jnp.sum(eG, axis=1, keepdims=True))
            ctx_h = (jnp.dot(eB.astype(jnp.bfloat16), v_ref[h, pl.ds(bs, BW), :],
                             preferred_element_type=jnp.float32)
                     + jnp.dot(eG.astype(jnp.bfloat16), v_ref[h, 0:GW, :],
                               preferred_element_type=jnp.float32)) / denom
            part = jnp.dot(ctx_h.astype(jnp.bfloat16),
                           wo16_ref[hs, :],
                           preferred_element_type=jnp.float32)
            acc = part if acc is None else acc + part
        acc_ref[pl.ds(r0, CHUNK), :] = acc

        @pl.when(c == 0)
        def _global_rows():
            q0 = (jnp.dot(x16_ref[0:32, :], wq16_ref[...],
                          preferred_element_type=jnp.float32)
                  * SCALE).astype(jnp.bfloat16)
            acc0 = None
            for h in range(H_PER):
                hs = slice(h * DH, (h + 1) * DH)
                s = lax.dot_general(q0[:, hs], k_ref[h],
                                    (((1,), (1,)), ((), ())),
                                    preferred_element_type=jnp.float32)
                e = jnp.exp(s)
                ctx_h = jnp.dot(e.astype(jnp.bfloat16), v_ref[h],
                                preferred_element_type=jnp.float32
                                ) / jnp.sum(e, axis=1, keepdims=True)
                p = jnp.dot(ctx_h.astype(jnp.bfloat16),
                            wo16_ref[hs, :],
                            preferred_element_type=jnp.float32)
                acc0 = p if acc0 is None else acc0 + p
            acc_ref[0:32, :] = acc0

    def hop_rdma(hop, fwd):
        s_slot = hop % N_DEV
        r_slot = (hop + 1) % N_DEV
        comm = comm_f if fwd else comm_b
        return pltpu.make_async_remote_copy(
            src_ref=comm.at[s_slot], dst_ref=comm.at[r_slot],
            send_sem=(send_f if fwd else send_b).at[hop],
            recv_sem=(recv_f if fwd else recv_b).at[hop],
            device_id=(right if fwd else left,),
            device_id_type=pl.DeviceIdType.MESH,
        )

    def rs_accum(hop, fwd):
        r_slot = (hop + 1) % N_DEV
        if fwd:
            c = lax.rem(my + (2 * N_DEV - hop - 1), N_DEV)
            comm_f[r_slot, :, :] = (
                comm_f[r_slot, :, :].astype(jnp.float32)
                + acc_ref[pl.ds(c * CHUNK, CHUNK), 0:HALF]
            ).astype(jnp.bfloat16)
        else:
            c = lax.rem(my + hop + 1, N_DEV)
            comm_b[r_slot, :, :] = (
                comm_b[r_slot, :, :].astype(jnp.float32)
                + acc_ref[pl.ds(c * CHUNK, CHUNK), HALF:DM]
            ).astype(jnp.bfloat16)
        return c, r_slot

    def out_store(c, slot, fwd):
        if fwd:
            out_ref[pl.ds(c * CHUNK, CHUNK), 0:HALF] = (
                comm_f[slot, :, :].astype(jnp.float32))
        else:
            out_ref[pl.ds(c * CHUNK, CHUNK), HALF:DM] = (
                comm_b[slot, :, :].astype(jnp.float32))

    c_my = my
    c_m1 = lax.rem(my + N_DEV - 1, N_DEV)
    c_p1 = lax.rem(my + 1, N_DEV)
    c_m2 = lax.rem(my + N_DEV - 2, N_DEV)

    import os
    if os.environ.get("NO_COMM"):
        compute_chunk(c_my)
        compute_chunk(c_m1)
        compute_chunk(c_p1)
        compute_chunk(c_m2)
        pl.semaphore_wait(barrier_sem, 2)
        out_ref[...] = acc_ref[...]
        return

    compute_chunk(c_my)
    pl.semaphore_wait(barrier_sem, 2)
    comm_f[0, :, :] = acc_ref[pl.ds(my * CHUNK, CHUNK), 0:HALF].astype(jnp.bfloat16)
    f0 = hop_rdma(0, True)
    f0.start()
    comm_b[0, :, :] = acc_ref[pl.ds(my * CHUNK, CHUNK), HALF:DM].astype(jnp.bfloat16)
    b0 = hop_rdma(0, False)
    b0.start()

    compute_chunk(c_m1)
    f0.wait()
    rs_accum(0, True)
    f1 = hop_rdma(1, True)
    f1.start()

    compute_chunk(c_p1)
    b0.wait()
    rs_accum(0, False)
    b1 = hop_rdma(1, False)
    b1.start()

    compute_chunk(c_m2)
    f1.wait()
    rs_accum(1, True)
    f2 = hop_rdma(2, True)
    f2.start()
    b1.wait()
    rs_accum(1, False)
    b2 = hop_rdma(2, False)
    b2.start()

    f2.wait()
    cf, sf = rs_accum(2, True)
    fa0 = hop_rdma(3, True)
    fa0.start()
    out_store(cf, sf, True)
    b2.wait()
    cb, sb = rs_accum(2, False)
    ba0 = hop_rdma(3, False)
    ba0.start()
    out_store(cb, sb, False)

    fa0.wait()
    fa1 = hop_rdma(4, True)
    fa1.start()
    out_store(my, 0, True)
    ba0.wait()
    ba1 = hop_rdma(4, False)
    ba1.start()
    out_store(my, 0, False)

    fa1.wait()
    fa2 = hop_rdma(5, True)
    fa2.start()
    out_store(c_m1, 1, True)
    ba1.wait()
    ba2 = hop_rdma(5, False)
    ba2.start()
    out_store(c_p1, 1, False)

    fa2.wait()
    out_store(c_m2, 2, True)
    ba2.wait()
    out_store(c_m2, 2, False)


def kernel(x, Wq, K_ext, V_ext, Wo):
    x2 = x[0]
    k = jnp.transpose(K_ext[0], (1, 0, 2)).astype(jnp.bfloat16)
    v = jnp.transpose(V_ext[0], (1, 0, 2)).astype(jnp.bfloat16)

    out = pl.pallas_call(
        _body,
        out_shape=jax.ShapeDtypeStruct((SQ, DM), jnp.float32),
        in_specs=[
            pl.BlockSpec(memory_space=pltpu.VMEM),
            pl.BlockSpec(memory_space=pl.ANY),
            pl.BlockSpec(memory_space=pltpu.VMEM),
            pl.BlockSpec(memory_space=pltpu.VMEM),
            pl.BlockSpec(memory_space=pl.ANY),
        ],
        out_specs=pl.BlockSpec(memory_space=pltpu.VMEM),
        scratch_shapes=[
            pltpu.VMEM((SQ, DM), jnp.float32),
            pltpu.VMEM((SQ, DM), jnp.bfloat16),
            pltpu.VMEM((DM, HD), jnp.float32),
            pltpu.VMEM((DM, HD), jnp.bfloat16),
            pltpu.VMEM((HD, DM), jnp.float32),
            pltpu.VMEM((HD, DM), jnp.bfloat16),
            pltpu.VMEM((N_DEV, CHUNK, HALF), jnp.bfloat16),
            pltpu.VMEM((N_DEV, CHUNK, HALF), jnp.bfloat16),
            pltpu.SemaphoreType.DMA((N_HOPS,)),
            pltpu.SemaphoreType.DMA((N_HOPS,)),
            pltpu.SemaphoreType.DMA((N_HOPS,)),
            pltpu.SemaphoreType.DMA((N_HOPS,)),
            pltpu.SemaphoreType.DMA((2,)),
        ],
        compiler_params=pltpu.CompilerParams(collective_id=0),
    )(x2, Wq, k, v, Wo)
    return out[None]


# device time: 54602 ns/iter; 1.2319x vs baseline; 1.0017x over previous
import jax
import jax.numpy as jnp
from jax import lax
from jax.experimental import pallas as pl
from jax.experimental.pallas import tpu as pltpu

N_DEV = 4
SQ = 1024
SKV = 1024
H_PER = 8
DH = 128
HD = H_PER * DH
DM = 1024
CHUNK = SQ // N_DEV
SCALE = 0.08838834764831843
N_HOPS = 2 * (N_DEV - 1)


HALF = DM // 2
BW = 512
GW = 128


def _body(x_ref, wq_hbm, k_ref, v_ref, wo_hbm, out_ref,
          acc_ref, x16_ref,
          wq32_ref, wq16_ref, wo32_ref, wo16_ref,
          comm_f, comm_b,
          send_f, recv_f, send_b, recv_b, cp_sems):
    my = lax.axis_index("i")
    right = lax.rem(my + 1, N_DEV)
    left = lax.rem(my + N_DEV - 1, N_DEV)

    barrier_sem = pltpu.get_barrier_semaphore()
    for nbr in (left, right):
        pl.semaphore_signal(barrier_sem, inc=1, device_id=(nbr,),
                            device_id_type=pl.DeviceIdType.MESH)

    w_off = pl.multiple_of(my * HD, 256)
    wq_dma = pltpu.make_async_copy(
        wq_hbm.at[:, pl.ds(w_off, HD)], wq32_ref, cp_sems.at[0])
    wo_dma = pltpu.make_async_copy(
        wo_hbm.at[pl.ds(w_off, HD), :], wo32_ref, cp_sems.at[1])
    wq_dma.start()
    wo_dma.start()
    x16_ref[...] = x_ref[...].astype(jnp.bfloat16)
    wq_dma.wait()
    wq16_ref[...] = wq32_ref[...].astype(jnp.bfloat16)
    wo_dma.wait()
    wo16_ref[...] = wo32_ref[...].astype(jnp.bfloat16)

    def compute_chunk(c):
        r0 = c * CHUNK
        bs = jnp.minimum(jnp.maximum(r0 - 128, 0), SKV - BW)
        bs = pl.multiple_of(bs, 128)
        qc = (jnp.dot(x16_ref[pl.ds(r0, CHUNK), :], wq16_ref[...],
                      preferred_element_type=jnp.float32)
              * SCALE).astype(jnp.bfloat16)
        qi = lax.broadcasted_iota(jnp.int32, (CHUNK, BW), 0) + r0
        kiB = lax.broadcasted_iota(jnp.int32, (CHUNK, BW), 1) + bs
        mB = ((jnp.abs(qi - kiB) <= 128) | (kiB < 32) | (qi < 32)
              ).astype(jnp.float32)
        kiG = lax.broadcasted_iota(jnp.int32, (CHUNK, GW), 1)
        mG = ((kiG < 32) & (bs >= 128)).astype(jnp.float32)
        acc = None
        for h in range(H_PER):
            hs = slice(h * DH, (h + 1) * DH)
            qh = qc[:, hs]
            sB = lax.dot_general(qh, k_ref[h, pl.ds(bs, BW), :],
                                 (((1,), (1,)), ((), ())),
                                 preferred_element_type=jnp.float32)
            eB = jnp.exp(sB) * mB
            sG = lax.dot_general(qh, k_ref[h, 0:GW, :],
                                 (((1,), (1,)), ((), ())),
                                 preferred_element_type=jnp.float32)
            eG = jnp.exp(sG) * mG
            denom = (jnp.sum(eB, axis=1, keepdims=True)
                     + jnp.sum(eG, axis=1, keepdims=True))
            ctx_h = (jnp.dot(eB.astype(jnp.bfloat16), v_ref[h, pl.ds(bs, BW), :],
                             preferred_element_type=jnp.float32)
                     + jnp.dot(eG.astype(jnp.bfloat16), v_ref[h, 0:GW, :],
                               preferred_element_type=jnp.float32)) / denom
            part = jnp.dot(ctx_h.astype(jnp.bfloat16),
                           wo16_ref[hs, :],
                           preferred_element_type=jnp.float32)
            acc = part if acc is None else acc + part
        acc_ref[pl.ds(r0, CHUNK), :] = acc

        @pl.when(c == 0)
        def _global_rows():
            q0 = (jnp.dot(x16_ref[0:32, :], wq16_ref[...],
                          preferred_element_type=jnp.float32)
                  * SCALE).astype(jnp.bfloat16)
            acc0 = None
            for h in range(H_PER):
                hs = slice(h * DH, (h + 1) * DH)
                s = lax.dot_general(q0[:, hs], k_ref[h],
                                    (((1,), (1,)), ((), ())),
                                    preferred_element_type=jnp.float32)
                e = jnp.exp(s)
                ctx_h = jnp.dot(e.astype(jnp.bfloat16), v_ref[h],
                                preferred_element_type=jnp.float32
                                ) / jnp.sum(e, axis=1, keepdims=True)
                p = jnp.dot(ctx_h.astype(jnp.bfloat16),
                            wo16_ref[hs, :],
                            preferred_element_type=jnp.float32)
                acc0 = p if acc0 is None else acc0 + p
            acc_ref[0:32, :] = acc0

    def hop_rdma(hop, fwd):
        s_slot = hop % N_DEV
        r_slot = (hop + 1) % N_DEV
        comm = comm_f if fwd else comm_b
        return pltpu.make_async_remote_copy(
            src_ref=comm.at[s_slot], dst_ref=comm.at[r_slot],
            send_sem=(send_f if fwd else send_b).at[hop],
            recv_sem=(recv_f if fwd else recv_b).at[hop],
            device_id=(right if fwd else left,),
            device_id_type=pl.DeviceIdType.MESH,
        )

    def rs_accum(hop, fwd):
        r_slot = (hop + 1) % N_DEV
        if fwd:
            c = lax.rem(my + (2 * N_DEV - hop - 1), N_DEV)
            comm_f[r_slot, :, :] = (
                comm_f[r_slot, :, :].astype(jnp.float32)
                + acc_ref[pl.ds(c * CHUNK, CHUNK), 0:HALF]
            ).astype(jnp.bfloat16)
        else:
            c = lax.rem(my + hop + 1, N_DEV)
            comm_b[r_slot, :, :] = (
                comm_b[r_slot, :, :].astype(jnp.float32)
                + acc_ref[pl.ds(c * CHUNK, CHUNK), HALF:DM]
            ).astype(jnp.bfloat16)
        return c, r_slot

    def out_store(c, slot, fwd):
        if fwd:
            out_ref[pl.ds(c * CHUNK, CHUNK), 0:HALF] = (
                comm_f[slot, :, :].astype(jnp.float32))
        else:
            out_ref[pl.ds(c * CHUNK, CHUNK), HALF:DM] = (
                comm_b[slot, :, :].astype(jnp.float32))

    c_my = my
    c_m1 = lax.rem(my + N_DEV - 1, N_DEV)
    c_p1 = lax.rem(my + 1, N_DEV)
    c_m2 = lax.rem(my + N_DEV - 2, N_DEV)

    compute_chunk(c_my)
    pl.semaphore_wait(barrier_sem, 2)
    comm_f[0, :, :] = acc_ref[pl.ds(my * CHUNK, CHUNK), 0:HALF].astype(jnp.bfloat16)
    f0 = hop_rdma(0, True)
    f0.start()
    comm_b[0, :, :] = acc_ref[pl.ds(my * CHUNK, CHUNK), HALF:DM].astype(jnp.bfloat16)
    b0 = hop_rdma(0, False)
    b0.start()

    compute_chunk(c_m1)
    f0.wait()
    rs_accum(0, True)
    f1 = hop_rdma(1, True)
    f1.start()

    compute_chunk(c_p1)
    b0.wait()
    rs_accum(0, False)
    b1 = hop_rdma(1, False)
    b1.start()

    compute_chunk(c_m2)
    f1.wait()
    rs_accum(1, True)
    f2 = hop_rdma(2, True)
    f2.start()
    b1.wait()
    rs_accum(1, False)
    b2 = hop_rdma(2, False)
    b2.start()

    f2.wait()
    cf, sf = rs_accum(2, True)
    fa0 = hop_rdma(3, True)
    fa0.start()
    out_store(cf, sf, True)
    b2.wait()
    cb, sb = rs_accum(2, False)
    ba0 = hop_rdma(3, False)
    ba0.start()
    out_store(cb, sb, False)

    fa0.wait()
    fa1 = hop_rdma(4, True)
    fa1.start()
    out_store(my, 0, True)
    ba0.wait()
    ba1 = hop_rdma(4, False)
    ba1.start()
    out_store(my, 0, False)

    fa1.wait()
    fa2 = hop_rdma(5, True)
    fa2.start()
    out_store(c_m1, 1, True)
    ba1.wait()
    ba2 = hop_rdma(5, False)
    ba2.start()
    out_store(c_p1, 1, False)

    fa2.wait()
    out_store(c_m2, 2, True)
    ba2.wait()
    out_store(c_m2, 2, False)


def kernel(x, Wq, K_ext, V_ext, Wo):
    x2 = x[0]
    k = jnp.transpose(K_ext[0], (1, 0, 2)).astype(jnp.bfloat16)
    v = jnp.transpose(V_ext[0], (1, 0, 2)).astype(jnp.bfloat16)

    out = pl.pallas_call(
        _body,
        out_shape=jax.ShapeDtypeStruct((SQ, DM), jnp.float32),
        in_specs=[
            pl.BlockSpec(memory_space=pltpu.VMEM),
            pl.BlockSpec(memory_space=pl.ANY),
            pl.BlockSpec(memory_space=pltpu.VMEM),
            pl.BlockSpec(memory_space=pltpu.VMEM),
            pl.BlockSpec(memory_space=pl.ANY),
        ],
        out_specs=pl.BlockSpec(memory_space=pltpu.VMEM),
        scratch_shapes=[
            pltpu.VMEM((SQ, DM), jnp.float32),
            pltpu.VMEM((SQ, DM), jnp.bfloat16),
            pltpu.VMEM((DM, HD), jnp.float32),
            pltpu.VMEM((DM, HD), jnp.bfloat16),
            pltpu.VMEM((HD, DM), jnp.float32),
            pltpu.VMEM((HD, DM), jnp.bfloat16),
            pltpu.VMEM((N_DEV, CHUNK, HALF), jnp.bfloat16),
            pltpu.VMEM((N_DEV, CHUNK, HALF), jnp.bfloat16),
            pltpu.SemaphoreType.DMA((N_HOPS,)),
            pltpu.SemaphoreType.DMA((N_HOPS,)),
            pltpu.SemaphoreType.DMA((N_HOPS,)),
            pltpu.SemaphoreType.DMA((N_HOPS,)),
            pltpu.SemaphoreType.DMA((2,)),
        ],
        compiler_params=pltpu.CompilerParams(collective_id=0),
    )(x2, Wq, k, v, Wo)
    return out[None]
